# baseline (speedup 1.0000x reference)
import math

import jax
import jax.numpy as jnp
from jax import lax
from jax.experimental import pallas as pl
from jax.experimental.pallas import tpu as pltpu
from jax.experimental.pallas import tpu_sc as plsc

_N_CLASSES = 100
_K = 64
_B = 4096
_SCALE = 1.0 / (_B * math.sqrt(768.0))

_NT = 16
_RPT = _B // _NT
_NG = _RPT // 16
_NIDX = 2


def _body(flat_hbm, lab_hbm, conf_hbm, out_hbm, parts_hbm,
          lab_v, conf_v, idx_v, rows_v, acc_v, out_v, red_v, sem):
    sid = lax.axis_index("s")
    base = sid * _RPT

    pltpu.sync_copy(lab_hbm.at[pl.ds(base, _RPT)], lab_v)
    pltpu.sync_copy(conf_hbm.at[pl.ds(base, _RPT)], conf_v)

    lanes = lax.broadcasted_iota(jnp.int32, (16,), 0)

    for g in range(_NG):
        lv = lab_v[pl.ds(g * 16, 16)]
        idx = (base + g * 16 + lanes) * _N_CLASSES + lv
        idx_v[g // 8, pl.ds((g % 8) * 16, 16)] = idx >> 1

    for j in range(_NIDX):
        pltpu.async_copy(flat_hbm.at[idx_v.at[j]],
                         rows_v.at[pl.ds(j * 128, 128)], sem)
    for j in range(_NIDX):
        pltpu.make_async_copy(flat_hbm.at[idx_v.at[j]],
                              rows_v.at[pl.ds(j * 128, 128)], sem).wait()

    acc = jnp.zeros((16,), jnp.float32)
    for g in range(_NG):
        confc = conf_v[pl.ds(g * 16, 16)]
        lv = lab_v[pl.ds(g * 16, 16)]
        par = ((base + g * 16 + lanes) * _N_CLASSES + lv) & 1
        for i in range(16):
            r = g * 16 + i
            mlo = rows_v[r, pl.ds(0, 16)]
            mhi = rows_v[r, pl.ds(64, 16)]
            for c in range(1, _K // 16):
                mlo = jnp.minimum(mlo, rows_v[r, pl.ds(c * 16, 16)])
                mhi = jnp.minimum(mhi, rows_v[r, pl.ds(64 + c * 16, 16)])
            m = jnp.where(par[i] == 1, mhi, mlo)
            for sh in (8, 4, 2, 1):
                m = jnp.minimum(
                    m, m.at[lanes ^ sh].get(mode="promise_in_bounds"))
            acc = acc + jnp.where(lanes == i, m, 0.0) * confc
    acc_v[...] = acc

    pltpu.sync_copy(acc_v, parts_hbm.at[sid])
    plsc.subcore_barrier()

    @pl.when(sid == 0)
    def _():
        pltpu.sync_copy(parts_hbm, red_v)
        s = red_v[0]
        for i in range(1, _NT):
            s = s + red_v[i]
        for sh in (8, 4, 2, 1):
            s = s + s.at[lanes ^ sh].get(mode="promise_in_bounds")
        out_v[...] = s * _SCALE
        pltpu.sync_copy(out_v, out_hbm)


_sc_call = pl.kernel(
    _body,
    out_type=(jax.ShapeDtypeStruct((16,), jnp.float32),
              jax.ShapeDtypeStruct((_NT, 16), jnp.float32)),
    mesh=plsc.VectorSubcoreMesh(core_axis_name="c", subcore_axis_name="s",
                                num_cores=1),
    scratch_types=[
        pltpu.VMEM((_RPT,), jnp.int32),
        pltpu.VMEM((_RPT,), jnp.float32),
        pltpu.VMEM((_NIDX, 128), jnp.int32),
        pltpu.VMEM((_RPT, 128), jnp.float32),
        pltpu.VMEM((16,), jnp.float32),
        pltpu.VMEM((16,), jnp.float32),
        pltpu.VMEM((_NT, 16), jnp.float32),
        pltpu.SemaphoreType.DMA,
    ],
    compiler_params=pltpu.CompilerParams(needs_layout_passes=False,
                                         use_tc_tiling_on_sc=True),
)


@jax.jit
def _center_loss(distances, labels, confidence):
    flat = distances.reshape(_B * _N_CLASSES // 2, 2 * _K)
    out, _ = _sc_call(flat, labels.astype(jnp.int32), confidence)
    return out[0]


def kernel(distances, labels, confidence):
    return _center_loss(distances, labels, confidence)


# speedup vs baseline: 1.4453x; 1.4453x over previous
import math

import jax
import jax.numpy as jnp
from jax import lax
from jax.experimental import pallas as pl
from jax.experimental.pallas import tpu as pltpu
from jax.experimental.pallas import tpu_sc as plsc

_N_CLASSES = 100
_K = 64
_B = 4096
_SCALE = 1.0 / (_B * math.sqrt(768.0))

_NT = 16
_RPT = _B // _NT
_NG = _RPT // 16
_FIRE = 16


def _body(dist_hbm, lab_hbm, conf_hbm, out_hbm, parts_hbm,
          lab_v, conf_v, rows_v, mbuf_v, acc_v, out_v, red_v, sem):
    sid = lax.axis_index("s")
    base = sid * _RPT

    pltpu.sync_copy(lab_hbm.at[pl.ds(base, _RPT)], lab_v)
    pltpu.sync_copy(conf_hbm.at[pl.ds(base, _RPT)], conf_v)

    lanes = lax.broadcasted_iota(jnp.int32, (16,), 0)

    for g in range(_NG):
        lv = lab_v[pl.ds(g * 16, 16)]
        for i in range(_FIRE):
            r = g * 16 + i
            pltpu.async_copy(dist_hbm.at[base + r, lv[i]], rows_v.at[r], sem)
        for i in range(_FIRE):
            r = g * 16 + i
            pltpu.make_async_copy(dist_hbm.at[base + r, lv[i]],
                                  rows_v.at[r], sem).wait()

    acc = jnp.zeros((16,), jnp.float32)
    for g in range(_NG):
        for i in range(16):
            r = g * 16 + i
            m = rows_v[r, pl.ds(0, 16)]
            for c in range(1, _K // 16):
                m = jnp.minimum(m, rows_v[r, pl.ds(c * 16, 16)])
            mbuf_v[pl.ds(i * 16, 16)] = m
        tidx = lanes * 16
        rowmin = plsc.load_gather(mbuf_v, [tidx])
        for j in range(1, 16):
            rowmin = jnp.minimum(rowmin, plsc.load_gather(mbuf_v, [tidx + j]))
        acc = acc + rowmin * conf_v[pl.ds(g * 16, 16)]
    acc_v[...] = acc

    pltpu.sync_copy(acc_v, parts_hbm.at[sid])
    plsc.subcore_barrier()

    @pl.when(sid == 0)
    def _():
        pltpu.sync_copy(parts_hbm, red_v)
        s = red_v[0]
        for i in range(1, _NT):
            s = s + red_v[i]
        total = jnp.sum(s) * _SCALE
        out_v[...] = jnp.full((16,), total, jnp.float32)
        pltpu.sync_copy(out_v, out_hbm)


_sc_call = pl.kernel(
    _body,
    out_type=(jax.ShapeDtypeStruct((16,), jnp.float32),
              jax.ShapeDtypeStruct((_NT, 16), jnp.float32)),
    mesh=plsc.VectorSubcoreMesh(core_axis_name="c", subcore_axis_name="s",
                                num_cores=1),
    scratch_types=[
        pltpu.VMEM((_RPT,), jnp.int32),
        pltpu.VMEM((_RPT,), jnp.float32),
        pltpu.VMEM((_RPT, _K), jnp.float32),
        pltpu.VMEM((256,), jnp.float32),
        pltpu.VMEM((16,), jnp.float32),
        pltpu.VMEM((16,), jnp.float32),
        pltpu.VMEM((_NT, 16), jnp.float32),
        pltpu.SemaphoreType.DMA,
    ],
    compiler_params=pltpu.CompilerParams(needs_layout_passes=False,
                                         use_tc_tiling_on_sc=True),
)


@jax.jit
def _center_loss(distances, labels, confidence):
    out, _ = _sc_call(distances, labels.astype(jnp.int32), confidence)
    return out[0]


def kernel(distances, labels, confidence):
    return _center_loss(distances, labels, confidence)


# speedup vs baseline: 1.5073x; 1.0429x over previous
import math

import jax
import jax.numpy as jnp
from jax import lax
from jax.experimental import pallas as pl
from jax.experimental.pallas import tpu as pltpu
from jax.experimental.pallas import tpu_sc as plsc

_N_CLASSES = 100
_K = 64
_B = 4096
_SCALE = 1.0 / (_B * math.sqrt(768.0))

_NT = 16
_RPT = _B // _NT
_NG = _RPT // 16
_FIRE = 16


def _body(dist_hbm, lab_hbm, conf_hbm, out_hbm, parts_hbm,
          lab_v, conf_v, rows_v, mbuf_v, acc_v, out_v, red_v, sem):
    sid = lax.axis_index("s")
    base = sid * _RPT

    pltpu.sync_copy(lab_hbm.at[pl.ds(base, _RPT)], lab_v)
    pltpu.sync_copy(conf_hbm.at[pl.ds(base, _RPT)], conf_v)

    lanes = lax.broadcasted_iota(jnp.int32, (16,), 0)

    def _fire(g):
        lv = lab_v[pl.ds(g * 16, 16)]
        for i in range(_FIRE):
            r = g * 16 + i
            pltpu.async_copy(dist_hbm.at[base + r, lv[i]], rows_v.at[r], sem)

    def _wait(g):
        lv = lab_v[pl.ds(g * 16, 16)]
        for i in range(_FIRE):
            r = g * 16 + i
            pltpu.make_async_copy(dist_hbm.at[base + r, lv[i]],
                                  rows_v.at[r], sem).wait()

    _fire(0)
    acc = jnp.zeros((16,), jnp.float32)
    for g in range(_NG):
        if g + 1 < _NG:
            _fire(g + 1)
        _wait(g)
        for i in range(16):
            r = g * 16 + i
            m = rows_v[r, pl.ds(0, 16)]
            for c in range(1, _K // 16):
                m = jnp.minimum(m, rows_v[r, pl.ds(c * 16, 16)])
            mbuf_v[pl.ds(i * 16, 16)] = m
        tidx = lanes * 16
        rowmin = plsc.load_gather(mbuf_v, [tidx])
        for j in range(1, 16):
            rowmin = jnp.minimum(rowmin, plsc.load_gather(mbuf_v, [tidx + j]))
        acc = acc + rowmin * conf_v[pl.ds(g * 16, 16)]
    acc_v[...] = acc

    pltpu.sync_copy(acc_v, parts_hbm.at[sid])
    plsc.subcore_barrier()

    @pl.when(sid == 0)
    def _():
        pltpu.sync_copy(parts_hbm, red_v)
        s = red_v[0]
        for i in range(1, _NT):
            s = s + red_v[i]
        total = jnp.sum(s) * _SCALE
        out_v[...] = jnp.full((16,), total, jnp.float32)
        pltpu.sync_copy(out_v, out_hbm)


_sc_call = pl.kernel(
    _body,
    out_type=(jax.ShapeDtypeStruct((16,), jnp.float32),
              jax.ShapeDtypeStruct((_NT, 16), jnp.float32)),
    mesh=plsc.VectorSubcoreMesh(core_axis_name="c", subcore_axis_name="s",
                                num_cores=1),
    scratch_types=[
        pltpu.VMEM((_RPT,), jnp.int32),
        pltpu.VMEM((_RPT,), jnp.float32),
        pltpu.VMEM((_RPT, _K), jnp.float32),
        pltpu.VMEM((256,), jnp.float32),
        pltpu.VMEM((16,), jnp.float32),
        pltpu.VMEM((16,), jnp.float32),
        pltpu.VMEM((_NT, 16), jnp.float32),
        pltpu.SemaphoreType.DMA,
    ],
    compiler_params=pltpu.CompilerParams(needs_layout_passes=False,
                                         use_tc_tiling_on_sc=True),
)


@jax.jit
def _center_loss(distances, labels, confidence):
    out, _ = _sc_call(distances, labels.astype(jnp.int32), confidence)
    return out[0]


def kernel(distances, labels, confidence):
    return _center_loss(distances, labels, confidence)
